# baseline (speedup 1.0000x reference)
import jax
import jax.numpy as jnp
from jax.experimental import pallas as pl
from jax.experimental.pallas import tpu as pltpu

B, S, D, H, K = 1, 2048, 1024, 16, 16
DH = D // H
QB = 128
NEG = -1e9


def _mega_body(x_ref, w_ref, cq_ref, ct_ref, out_ref, metric_ref,
               qs, ks, vs, bs, ms):
    p = pl.program_id(0)
    i = pl.program_id(1)
    rows = pl.ds(i * QB, QB)

    @pl.when(p == 0)
    def _proj_and_knn():
        y = jnp.dot(x_ref[...], w_ref[...], preferred_element_type=jnp.float32)
        scale = 1.0 / jnp.sqrt(jnp.float32(DH))
        qs[rows, :] = (y[:, 0 * D:1 * D] * scale).astype(jnp.bfloat16)
        kk = y[:, 1 * D:2 * D]
        ks[rows, :] = kk.astype(jnp.bfloat16)
        vs[rows, :] = y[:, 2 * D:3 * D].astype(jnp.bfloat16)
        acc = kk[:, 0:DH]
        for h in range(1, H):
            acc = acc + kk[:, h * DH:(h + 1) * DH]
        ms[rows, :] = acc * (1.0 / H)

        cq = cq_ref[...]
        dx = cq[:, 0:1] - ct_ref[0:1, :]
        dy = cq[:, 1:2] - ct_ref[1:2, :]
        dz = cq[:, 2:3] - ct_ref[2:3, :]
        d0 = jnp.sqrt(dx * dx + dy * dy + dz * dz)

        d = d0
        inmask = jnp.zeros((QB, S), jnp.bool_)
        for _ in range(K + 1):
            m = jnp.min(d, axis=1, keepdims=True)
            sel = d == m
            inmask = inmask | sel
            d = jnp.where(sel, jnp.inf, d)
        cnt = jnp.sum(inmask.astype(jnp.float32), axis=1)
        sel0 = d0 == jnp.min(d0, axis=1, keepdims=True)
        bs[rows, :] = jnp.where(
            inmask & jnp.logical_not(sel0), 0.0, NEG).astype(jnp.bfloat16)

        @pl.when(jnp.logical_not(jnp.all(cnt == float(K + 1))))
        def _slow_exact():
            dd = d0
            iota = jax.lax.broadcasted_iota(jnp.int32, (QB, S), 1)
            accb = jnp.full((QB, S), NEG, jnp.float32)
            for it in range(K + 1):
                mm = jnp.min(dd, axis=1, keepdims=True)
                amin = jnp.min(jnp.where(dd == mm, iota, S),
                               axis=1, keepdims=True)
                s2 = iota == amin
                if it > 0:
                    accb = jnp.where(s2, 0.0, accb)
                dd = jnp.where(s2, jnp.inf, dd)
            bs[rows, :] = accb.astype(jnp.bfloat16)

    @pl.when(p == 1)
    def _attn():
        bias = bs[rows, :].astype(jnp.float32)
        for h in range(H):
            qh = qs[rows, h * DH:(h + 1) * DH]
            kh = ks[:, h * DH:(h + 1) * DH]
            vh = vs[:, h * DH:(h + 1) * DH]
            s = jax.lax.dot_general(qh, kh, (((1,), (1,)), ((), ())),
                                    preferred_element_type=jnp.float32)
            e = jnp.exp(s + bias)
            eb = e.astype(jnp.bfloat16)
            den = jnp.sum(e, axis=1, keepdims=True)
            ov = jnp.dot(eb, vh, preferred_element_type=jnp.float32)
            out_ref[:, h * DH:(h + 1) * DH] = ov / den
        metric_ref[...] = ms[rows, :]


@jax.jit
def _run(x, coords, Wq, Wk, Wv):
    x2 = x[0].astype(jnp.bfloat16)
    c2 = coords[0]
    cT = c2.T
    w_all = jnp.concatenate([Wq.T, Wk.T, Wv.T], axis=1).astype(jnp.bfloat16)

    out, metric = pl.pallas_call(
        _mega_body,
        grid=(2, S // QB),
        in_specs=[
            pl.BlockSpec((QB, D), lambda p, i: (i * (1 - p), 0)),
            pl.BlockSpec((D, 3 * D), lambda p, i: (0, 0)),
            pl.BlockSpec((QB, 3), lambda p, i: (i * (1 - p), 0)),
            pl.BlockSpec((3, S), lambda p, i: (0, 0)),
        ],
        out_specs=[
            pl.BlockSpec((QB, D), lambda p, i: (i * p, 0)),
            pl.BlockSpec((QB, DH), lambda p, i: (i * p, 0)),
        ],
        out_shape=[
            jax.ShapeDtypeStruct((S, D), jnp.float32),
            jax.ShapeDtypeStruct((S, DH), jnp.float32),
        ],
        scratch_shapes=[
            pltpu.VMEM((S, D), jnp.bfloat16),
            pltpu.VMEM((S, D), jnp.bfloat16),
            pltpu.VMEM((S, D), jnp.bfloat16),
            pltpu.VMEM((S, S), jnp.bfloat16),
            pltpu.VMEM((S, DH), jnp.float32),
        ],
    )(x2, w_all, c2, cT)

    return out[None], metric[None]


def kernel(x, coords, Wq, Wk, Wv):
    return _run(x, coords, Wq, Wk, Wv)


# speedup vs baseline: 1.0252x; 1.0252x over previous
import functools

import jax
import jax.numpy as jnp
from jax.experimental import pallas as pl
from jax.experimental.pallas import tpu as pltpu

B, S, D, H, K = 1, 2048, 1024, 16, 16
DH = D // H
QB = 256
KB = 256
NEG = -1e9


def _proj_knn_body(x_ref, w_ref, cq_ref, ct_ref,
                   q_ref, k_ref, v_ref, m_ref, mask_ref):
    y = jnp.dot(x_ref[...], w_ref[...], preferred_element_type=jnp.float32)
    scale = 1.0 / jnp.sqrt(jnp.float32(DH))
    q_ref[...] = (y[:, 0 * D:1 * D] * scale).astype(jnp.bfloat16)
    kk = y[:, 1 * D:2 * D]
    k_ref[...] = kk.astype(jnp.bfloat16)
    v_ref[...] = y[:, 2 * D:3 * D].astype(jnp.bfloat16)
    acc = kk[:, 0:DH]
    for h in range(1, H):
        acc = acc + kk[:, h * DH:(h + 1) * DH]
    m_ref[...] = acc * (1.0 / H)
    _knn_into(cq_ref, ct_ref, mask_ref)


def _knn_into(cq_ref, ct_ref, mask_ref):
    cq = cq_ref[...]
    dx = cq[:, 0:1] - ct_ref[0:1, :]
    dy = cq[:, 1:2] - ct_ref[1:2, :]
    dz = cq[:, 2:3] - ct_ref[2:3, :]
    d0 = jnp.sqrt(dx * dx + dy * dy + dz * dz)

    d = d0
    inmask = jnp.zeros((KB, S), jnp.bool_)
    for _ in range(K + 1):
        m = jnp.min(d, axis=1, keepdims=True)
        sel = d == m
        inmask = inmask | sel
        d = jnp.where(sel, jnp.inf, d)
    cnt = jnp.sum(inmask.astype(jnp.float32), axis=1)
    sel0 = d0 == jnp.min(d0, axis=1, keepdims=True)
    mask_ref[...] = jnp.where(
        inmask & jnp.logical_not(sel0), 0.0, NEG).astype(jnp.bfloat16)

    @pl.when(jnp.logical_not(jnp.all(cnt == float(K + 1))))
    def _slow_exact():
        dd = d0
        iota = jax.lax.broadcasted_iota(jnp.int32, (KB, S), 1)
        sel_acc = jnp.full((KB, S), NEG, jnp.float32)
        for it in range(K + 1):
            m = jnp.min(dd, axis=1, keepdims=True)
            amin = jnp.min(jnp.where(dd == m, iota, S), axis=1, keepdims=True)
            sel = iota == amin
            if it > 0:
                sel_acc = jnp.where(sel, 0.0, sel_acc)
            dd = jnp.where(sel, jnp.inf, dd)
        mask_ref[...] = sel_acc.astype(jnp.bfloat16)


def _attn_body(q_ref, k_ref, v_ref, m_ref, o_ref):
    bias = m_ref[...].astype(jnp.float32)
    for h in range(H):
        qh = q_ref[:, h * DH:(h + 1) * DH]
        kh = k_ref[:, h * DH:(h + 1) * DH]
        vh = v_ref[:, h * DH:(h + 1) * DH]
        s = jax.lax.dot_general(qh, kh, (((1,), (1,)), ((), ())),
                                preferred_element_type=jnp.float32)
        e = jnp.exp(s + bias)
        eb = e.astype(jnp.bfloat16)
        den = jnp.sum(e, axis=1, keepdims=True)
        ov = jnp.dot(eb, vh, preferred_element_type=jnp.float32)
        o_ref[:, h * DH:(h + 1) * DH] = ov / den


@jax.jit
def _run(x, coords, Wq, Wk, Wv):
    x2 = x[0].astype(jnp.bfloat16)
    c2 = coords[0]
    cT = c2.T
    w_all = jnp.concatenate([Wq.T, Wk.T, Wv.T], axis=1).astype(jnp.bfloat16)

    q, k, v, metric, mask = pl.pallas_call(
        _proj_knn_body,
        grid=(S // QB,),
        in_specs=[
            pl.BlockSpec((QB, D), lambda i: (i, 0)),
            pl.BlockSpec((D, 3 * D), lambda i: (0, 0)),
            pl.BlockSpec((QB, 3), lambda i: (i, 0)),
            pl.BlockSpec((3, S), lambda i: (0, 0)),
        ],
        out_specs=[
            pl.BlockSpec((QB, D), lambda i: (i, 0)),
            pl.BlockSpec((QB, D), lambda i: (i, 0)),
            pl.BlockSpec((QB, D), lambda i: (i, 0)),
            pl.BlockSpec((QB, DH), lambda i: (i, 0)),
            pl.BlockSpec((QB, S), lambda i: (i, 0)),
        ],
        out_shape=[
            jax.ShapeDtypeStruct((S, D), jnp.bfloat16),
            jax.ShapeDtypeStruct((S, D), jnp.bfloat16),
            jax.ShapeDtypeStruct((S, D), jnp.bfloat16),
            jax.ShapeDtypeStruct((S, DH), jnp.float32),
            jax.ShapeDtypeStruct((S, S), jnp.bfloat16),
        ],
    )(x2, w_all, c2, cT)

    out = pl.pallas_call(
        _attn_body,
        grid=(S // QB,),
        in_specs=[
            pl.BlockSpec((QB, D), lambda i: (i, 0)),
            pl.BlockSpec((S, D), lambda i: (0, 0)),
            pl.BlockSpec((S, D), lambda i: (0, 0)),
            pl.BlockSpec((QB, S), lambda i: (i, 0)),
        ],
        out_specs=pl.BlockSpec((QB, D), lambda i: (i, 0)),
        out_shape=jax.ShapeDtypeStruct((S, D), jnp.float32),
    )(q, k, v, mask)

    return out[None], metric[None]


def kernel(x, coords, Wq, Wk, Wv):
    return _run(x, coords, Wq, Wk, Wv)


# speedup vs baseline: 1.0283x; 1.0031x over previous
import jax
import jax.numpy as jnp
from jax.experimental import pallas as pl
from jax.experimental.pallas import tpu as pltpu

B, S, D, H, K = 1, 2048, 1024, 16, 16
DH = D // H
QB = 256
KB = 256
NEG = -1e9


def _proj_knn_body(x_ref, w_ref, cq_ref, ct_ref,
                   q_ref, k_ref, v_ref, m_ref, mask_ref):
    y = jnp.dot(x_ref[...], w_ref[...], preferred_element_type=jnp.float32)
    scale = 1.0 / jnp.sqrt(jnp.float32(DH))
    q_ref[...] = (y[:, 0 * D:1 * D] * scale).astype(jnp.bfloat16)
    kk = y[:, 1 * D:2 * D]
    k_ref[...] = kk.astype(jnp.bfloat16)
    v_ref[...] = y[:, 2 * D:3 * D].astype(jnp.bfloat16)
    acc = kk[:, 0:DH]
    for h in range(1, H):
        acc = acc + kk[:, h * DH:(h + 1) * DH]
    m_ref[...] = acc * (1.0 / H)
    _knn_into(cq_ref, ct_ref, mask_ref)


def _knn_into(cq_ref, ct_ref, mask_ref):
    cq = cq_ref[...]
    dx = cq[:, 0:1] - ct_ref[0:1, :]
    dy = cq[:, 1:2] - ct_ref[1:2, :]
    dz = cq[:, 2:3] - ct_ref[2:3, :]
    d0 = jnp.sqrt(dx * dx + dy * dy + dz * dz)

    d = d0
    inmask = jnp.zeros((KB, S), jnp.bool_)
    for _ in range(K + 1):
        m = jnp.min(d, axis=1, keepdims=True)
        sel = d == m
        inmask = inmask | sel
        d = jnp.where(sel, jnp.inf, d)
    cnt = jnp.sum(inmask.astype(jnp.float32), axis=1)
    sel0 = d0 == jnp.min(d0, axis=1, keepdims=True)
    mask_ref[...] = jnp.where(
        inmask & jnp.logical_not(sel0), 0.0, NEG).astype(jnp.bfloat16)

    @pl.when(jnp.logical_not(jnp.all(cnt == float(K + 1))))
    def _slow_exact():
        dd = d0
        iota = jax.lax.broadcasted_iota(jnp.int32, (KB, S), 1)
        sel_acc = jnp.full((KB, S), NEG, jnp.float32)
        for it in range(K + 1):
            m = jnp.min(dd, axis=1, keepdims=True)
            amin = jnp.min(jnp.where(dd == m, iota, S), axis=1, keepdims=True)
            sel = iota == amin
            if it > 0:
                sel_acc = jnp.where(sel, 0.0, sel_acc)
            dd = jnp.where(sel, jnp.inf, dd)
        mask_ref[...] = sel_acc.astype(jnp.bfloat16)


def _attn_body(q_ref, k_ref, v_ref, m_ref, o_ref):
    bias = m_ref[...].astype(jnp.float32)
    for h in range(H):
        qh = q_ref[:, h * DH:(h + 1) * DH]
        kh = k_ref[:, h * DH:(h + 1) * DH]
        vh = v_ref[:, h * DH:(h + 1) * DH]
        s = jax.lax.dot_general(qh, kh, (((1,), (1,)), ((), ())),
                                preferred_element_type=jnp.float32)
        e = jnp.exp(s + bias)
        eb = e.astype(jnp.bfloat16)
        den = jnp.sum(e, axis=1, keepdims=True)
        ov = jnp.dot(eb, vh, preferred_element_type=jnp.float32)
        o_ref[:, h * DH:(h + 1) * DH] = ov / den


@jax.jit
def _run(x, coords, Wq, Wk, Wv):
    x2 = x[0].astype(jnp.bfloat16)
    c2 = coords[0]
    cT = c2.T
    w_all = jnp.concatenate([Wq.T, Wk.T, Wv.T], axis=1).astype(jnp.bfloat16)

    q, k, v, metric, mask = pl.pallas_call(
        _proj_knn_body,
        grid=(S // QB,),
        in_specs=[
            pl.BlockSpec((QB, D), lambda i: (i, 0)),
            pl.BlockSpec((D, 3 * D), lambda i: (0, 0)),
            pl.BlockSpec((QB, 3), lambda i: (i, 0)),
            pl.BlockSpec((3, S), lambda i: (0, 0)),
        ],
        out_specs=[
            pl.BlockSpec((QB, D), lambda i: (i, 0)),
            pl.BlockSpec((QB, D), lambda i: (i, 0)),
            pl.BlockSpec((QB, D), lambda i: (i, 0)),
            pl.BlockSpec((QB, DH), lambda i: (i, 0)),
            pl.BlockSpec((QB, S), lambda i: (i, 0)),
        ],
        out_shape=[
            jax.ShapeDtypeStruct((S, D), jnp.bfloat16),
            jax.ShapeDtypeStruct((S, D), jnp.bfloat16),
            jax.ShapeDtypeStruct((S, D), jnp.bfloat16),
            jax.ShapeDtypeStruct((S, DH), jnp.float32),
            jax.ShapeDtypeStruct((S, S), jnp.bfloat16),
        ],
    )(x2, w_all, c2, cT)

    out = pl.pallas_call(
        _attn_body,
        grid=(S // QB,),
        in_specs=[
            pl.BlockSpec((QB, D), lambda i: (i, 0)),
            pl.BlockSpec((S, D), lambda i: (0, 0)),
            pl.BlockSpec((S, D), lambda i: (0, 0)),
            pl.BlockSpec((QB, S), lambda i: (i, 0)),
        ],
        out_specs=pl.BlockSpec((QB, D), lambda i: (i, 0)),
        out_shape=jax.ShapeDtypeStruct((S, D), jnp.float32),
    )(q, k, v, mask)

    return out[None], metric[None]


def kernel(x, coords, Wq, Wk, Wv):
    return _run(x, coords, Wq, Wk, Wv)
